# initial kernel scaffold (unmeasured)
import jax
import jax.numpy as jnp
from jax import lax
from jax.experimental import pallas as pl
from jax.experimental.pallas import tpu as pltpu

N_DEV = 16


def kernel(x, W):
    m, _ = x.shape
    _, n_per = W.shape

    def body(x_ref, w_ref, out_ref, comm_ref, send_sems, recv_sems):
        my = lax.axis_index("i")
        left = lax.rem(my + N_DEV - 1, N_DEV)
        right = lax.rem(my + 1, N_DEV)

        barrier_sem = pltpu.get_barrier_semaphore()
        for nbr in (left, right):
            pl.semaphore_signal(
                barrier_sem, inc=1,
                device_id=(nbr,), device_id_type=pl.DeviceIdType.MESH,
            )
        pl.semaphore_wait(barrier_sem, 2)

        xb = x_ref[...].astype(jnp.bfloat16)
        wb = w_ref[...].astype(jnp.bfloat16)
        logits = jnp.dot(xb, wb, preferred_element_type=jnp.float32)
        comm_ref[0, :, :] = logits.astype(jnp.bfloat16)
        e = jnp.exp(logits)
        out_ref[:, pl.ds(my * n_per, n_per)] = e
        s = jnp.sum(e, axis=1, keepdims=True)

        for h in range(N_DEV - 1):
            send_slot = h % 2
            recv_slot = (h + 1) % 2
            rdma = pltpu.make_async_remote_copy(
                src_ref=comm_ref.at[send_slot],
                dst_ref=comm_ref.at[recv_slot],
                send_sem=send_sems.at[send_slot],
                recv_sem=recv_sems.at[recv_slot],
                device_id=(right,),
                device_id_type=pl.DeviceIdType.MESH,
            )
            rdma.start()
            rdma.wait()

            origin = lax.rem(my + N_DEV - h - 1, N_DEV)
            eh = jnp.exp(comm_ref[recv_slot, :, :].astype(jnp.float32))
            out_ref[:, pl.ds(origin * n_per, n_per)] = eh
            s = s + jnp.sum(eh, axis=1, keepdims=True)

        inv = 1.0 / s
        for j in range(N_DEV):
            sl = pl.ds(j * n_per, n_per)
            out_ref[:, sl] = out_ref[:, sl] * inv

    return pl.pallas_call(
        body,
        out_shape=jax.ShapeDtypeStruct((m, N_DEV * n_per), jnp.float32),
        in_specs=[
            pl.BlockSpec(memory_space=pltpu.VMEM),
            pl.BlockSpec(memory_space=pltpu.VMEM),
        ],
        out_specs=pl.BlockSpec(memory_space=pltpu.VMEM),
        scratch_shapes=[
            pltpu.VMEM((2, m, n_per), jnp.bfloat16),
            pltpu.SemaphoreType.DMA((2,)),
            pltpu.SemaphoreType.DMA((2,)),
        ],
        compiler_params=pltpu.CompilerParams(collective_id=0),
    )(x, W)


# baseline (device time: 425740 ns/iter reference)
import jax
import jax.numpy as jnp
from jax import lax
from jax.experimental import pallas as pl
from jax.experimental.pallas import tpu as pltpu

N_DEV = 16


def kernel(x, W):
    m, _ = x.shape
    _, n_per = W.shape

    def body(x_ref, w_ref, out_ref, comm_ref, stage_ref, sums_ref,
             send_sems, recv_sems, sums_send_sems, sums_recv_sems,
             copy_sems):
        my = lax.axis_index("i")
        right = lax.rem(my + 1, N_DEV)

        barrier_sem = pltpu.get_barrier_semaphore()
        for d in range(1, N_DEV):
            pl.semaphore_signal(
                barrier_sem, inc=1,
                device_id=(lax.rem(my + d, N_DEV),),
                device_id_type=pl.DeviceIdType.MESH,
            )
        pl.semaphore_wait(barrier_sem, N_DEV - 1)

        xb = x_ref[...].astype(jnp.bfloat16)
        wb = w_ref[...].astype(jnp.bfloat16)
        logits = jnp.dot(xb, wb, preferred_element_type=jnp.float32)
        lb = logits.astype(jnp.bfloat16)
        comm_ref[0, :, :] = lb

        rdma0 = pltpu.make_async_remote_copy(
            src_ref=comm_ref.at[0],
            dst_ref=comm_ref.at[1],
            send_sem=send_sems.at[0],
            recv_sem=recv_sems.at[1],
            device_id=(right,),
            device_id_type=pl.DeviceIdType.MESH,
        )
        rdma0.start()

        e_own = jnp.exp(lb.astype(jnp.float32))
        partial = jnp.sum(e_own, axis=1)
        sums_ref[pl.ds(my, 1), :] = partial[None, :]
        sums_rdmas = []
        for d in range(1, N_DEV):
            r = pltpu.make_async_remote_copy(
                src_ref=sums_ref.at[pl.ds(my, 1)],
                dst_ref=sums_ref.at[pl.ds(my, 1)],
                send_sem=sums_send_sems.at[d],
                recv_sem=sums_recv_sems.at[d],
                device_id=(lax.rem(my + d, N_DEV),),
                device_id_type=pl.DeviceIdType.MESH,
            )
            r.start()
            sums_rdmas.append(r)
        for r in sums_rdmas:
            r.wait_recv()
        total = jnp.sum(sums_ref[...], axis=0)
        inv = (1.0 / total)[:, None]

        stage_ref[2, :, :] = e_own * inv
        own_copy = pltpu.make_async_copy(
            stage_ref.at[2],
            out_ref.at[:, pl.ds(my * n_per, n_per)],
            copy_sems.at[2],
        )
        own_copy.start()
        pending = [None, None, own_copy]

        rdma = rdma0
        for h in range(N_DEV - 1):
            rdma.wait()
            recv_slot = (h + 1) % 2
            if h + 1 < N_DEV - 1:
                next_rdma = pltpu.make_async_remote_copy(
                    src_ref=comm_ref.at[recv_slot],
                    dst_ref=comm_ref.at[h % 2],
                    send_sem=send_sems.at[recv_slot],
                    recv_sem=recv_sems.at[h % 2],
                    device_id=(right,),
                    device_id_type=pl.DeviceIdType.MESH,
                )
                next_rdma.start()
            else:
                next_rdma = None

            origin = lax.rem(my + N_DEV - h - 1, N_DEV)
            eh = jnp.exp(comm_ref[recv_slot, :, :].astype(jnp.float32)) * inv
            slot = h % 2
            if pending[slot] is not None:
                pending[slot].wait()
            stage_ref[slot, :, :] = eh
            copy = pltpu.make_async_copy(
                stage_ref.at[slot],
                out_ref.at[:, pl.ds(origin * n_per, n_per)],
                copy_sems.at[slot],
            )
            copy.start()
            pending[slot] = copy
            rdma = next_rdma

        for c in pending:
            if c is not None:
                c.wait()
        for r in sums_rdmas:
            r.wait_send()

    return pl.pallas_call(
        body,
        out_shape=jax.ShapeDtypeStruct((m, N_DEV * n_per), jnp.float32),
        in_specs=[
            pl.BlockSpec(memory_space=pltpu.VMEM),
            pl.BlockSpec(memory_space=pltpu.VMEM),
        ],
        out_specs=pl.BlockSpec(memory_space=pl.ANY),
        scratch_shapes=[
            pltpu.VMEM((2, m, n_per), jnp.bfloat16),
            pltpu.VMEM((3, m, n_per), jnp.float32),
            pltpu.VMEM((N_DEV, m), jnp.float32),
            pltpu.SemaphoreType.DMA((2,)),
            pltpu.SemaphoreType.DMA((2,)),
            pltpu.SemaphoreType.DMA((N_DEV,)),
            pltpu.SemaphoreType.DMA((N_DEV,)),
            pltpu.SemaphoreType.DMA((3,)),
        ],
        compiler_params=pltpu.CompilerParams(collective_id=0),
    )(x, W)


# device time: 234429 ns/iter; 1.8161x vs baseline; 1.8161x over previous
import jax
import jax.numpy as jnp
from jax import lax
from jax.experimental import pallas as pl
from jax.experimental.pallas import tpu as pltpu

N_DEV = 16
SUB = 4
HOPS = 8
R_KS = {j: (0, 1, 2, 3) if j < 7 else (0, 1) for j in range(HOPS)}
L_KS = {j: (0, 1, 2, 3) if j < 7 else (2, 3) for j in range(HOPS)}


def kernel(x, W):
    m, _ = x.shape
    _, n_per = W.shape
    rr = m // SUB

    def body(x_ref, w_ref, out_ref, comm_r, comm_l, stage_ref, sums_ref,
             send_sems_r, recv_sems_r, send_sems_l, recv_sems_l,
             sums_send_sems, sums_recv_sems, copy_sems):
        my = lax.axis_index("i")

        zq = my // 4
        pq = lax.rem(my, 4)
        rp = 4 * zq + lax.rem(pq + zq, 4)

        def ring_at(pos):
            zr = pos // 4
            pr = lax.rem(pos, 4)
            return 4 * zr + lax.rem(pr - zr + 4, 4)

        right = ring_at(lax.rem(rp + 1, N_DEV))
        left = ring_at(lax.rem(rp + N_DEV - 1, N_DEV))

        barrier_sem = pltpu.get_barrier_semaphore()
        for d in range(1, N_DEV):
            pl.semaphore_signal(
                barrier_sem, inc=1,
                device_id=(lax.rem(my + d, N_DEV),),
                device_id_type=pl.DeviceIdType.MESH,
            )
        pl.semaphore_wait(barrier_sem, N_DEV - 1)

        xb = x_ref[...].astype(jnp.bfloat16)
        wb = w_ref[...].astype(jnp.bfloat16)
        logits = jnp.dot(xb, wb, preferred_element_type=jnp.float32)
        lb = logits.astype(jnp.bfloat16)
        comm_r[0, :, :] = lb
        comm_l[0, :, :] = lb

        def make_hop(comm, ssem, rsem, dev, j, k):
            ss, rs = j % 2, (j + 1) % 2
            rows = pl.ds(k * rr, rr)
            return pltpu.make_async_remote_copy(
                src_ref=comm.at[ss, rows, :],
                dst_ref=comm.at[rs, rows, :],
                send_sem=ssem.at[ss, k],
                recv_sem=rsem.at[rs, k],
                device_id=(dev,),
                device_id_type=pl.DeviceIdType.MESH,
            )

        def make_r(j, k):
            return make_hop(comm_r, send_sems_r, recv_sems_r, right, j, k)

        def make_l(j, k):
            return make_hop(comm_l, send_sems_l, recv_sems_l, left, j, k)

        hops_r = {0: {k: make_r(0, k) for k in R_KS[0]}}
        hops_l = {0: {k: make_l(0, k) for k in L_KS[0]}}
        unsent_r = {}
        unsent_l = {}
        for k, r in hops_r[0].items():
            r.start()
            unsent_r[(0, k)] = r
        for k, r in hops_l[0].items():
            r.start()
            unsent_l[(0, k)] = r

        e_own = jnp.exp(lb.astype(jnp.float32))
        partial = jnp.sum(e_own, axis=1)
        sums_ref[pl.ds(my, 1), :] = partial[None, :]
        sums_rdmas = []
        for d in range(1, N_DEV):
            r = pltpu.make_async_remote_copy(
                src_ref=sums_ref.at[pl.ds(my, 1)],
                dst_ref=sums_ref.at[pl.ds(my, 1)],
                send_sem=sums_send_sems.at[d],
                recv_sem=sums_recv_sems.at[d],
                device_id=(lax.rem(my + d, N_DEV),),
                device_id_type=pl.DeviceIdType.MESH,
            )
            r.start()
            sums_rdmas.append(r)
        for r in sums_rdmas:
            r.wait_recv()
        total = jnp.sum(sums_ref[...], axis=0)
        inv = (1.0 / total)[:, None]

        stage_ref[4, :, :] = e_own * inv
        own_copy = pltpu.make_async_copy(
            stage_ref.at[4],
            out_ref.at[:, pl.ds(my * n_per, n_per)],
            copy_sems.at[4],
        )
        own_copy.start()
        pending = [None, None, None, None, own_copy]

        def store(slot, origin):
            copy = pltpu.make_async_copy(
                stage_ref.at[slot],
                out_ref.at[:, pl.ds(origin * n_per, n_per)],
                copy_sems.at[slot],
            )
            copy.start()
            pending[slot] = copy

        for j in range(HOPS):
            if j + 1 < HOPS:
                hops_r[j + 1] = {}
                hops_l[j + 1] = {}
            for k in range(SUB):
                if k in R_KS[j]:
                    hops_r[j][k].wait_recv()
                if j + 1 < HOPS and k in R_KS[j + 1]:
                    if (j - 1, k) in unsent_r:
                        unsent_r.pop((j - 1, k)).wait_send()
                    r = make_r(j + 1, k)
                    r.start()
                    hops_r[j + 1][k] = r
                    unsent_r[(j + 1, k)] = r
                if k in L_KS[j]:
                    hops_l[j][k].wait_recv()
                if j + 1 < HOPS and k in L_KS[j + 1]:
                    if (j - 1, k) in unsent_l:
                        unsent_l.pop((j - 1, k)).wait_send()
                    r = make_l(j + 1, k)
                    r.start()
                    hops_l[j + 1][k] = r
                    unsent_l[(j + 1, k)] = r

            rs = (j + 1) % 2
            if j < HOPS - 1:
                o_r = ring_at(lax.rem(rp + N_DEV - j - 1, N_DEV))
                slot = j % 2
                eh = jnp.exp(comm_r[rs, :, :].astype(jnp.float32)) * inv
                if pending[slot] is not None:
                    pending[slot].wait()
                stage_ref[slot, :, :] = eh
                store(slot, o_r)

                o_l = ring_at(lax.rem(rp + j + 1, N_DEV))
                slot = 2 + j % 2
                eh = jnp.exp(comm_l[rs, :, :].astype(jnp.float32)) * inv
                if pending[slot] is not None:
                    pending[slot].wait()
                stage_ref[slot, :, :] = eh
                store(slot, o_l)
            else:
                o = ring_at(lax.rem(rp + 8, N_DEV))
                half = m // 2
                slot = 1
                top = jnp.exp(comm_r[rs, :half, :].astype(jnp.float32))
                bot = jnp.exp(comm_l[rs, half:, :].astype(jnp.float32))
                if pending[slot] is not None:
                    pending[slot].wait()
                stage_ref[slot, :half, :] = top * inv[:half]
                stage_ref[slot, half:, :] = bot * inv[half:]
                store(slot, o)

        for r in unsent_r.values():
            r.wait_send()
        for r in unsent_l.values():
            r.wait_send()
        for c in pending:
            if c is not None:
                c.wait()
        for r in sums_rdmas:
            r.wait_send()

    return pl.pallas_call(
        body,
        out_shape=jax.ShapeDtypeStruct((m, N_DEV * n_per), jnp.float32),
        in_specs=[
            pl.BlockSpec(memory_space=pltpu.VMEM),
            pl.BlockSpec(memory_space=pltpu.VMEM),
        ],
        out_specs=pl.BlockSpec(memory_space=pltpu.MemorySpace.HBM),
        scratch_shapes=[
            pltpu.VMEM((2, m, n_per), jnp.bfloat16),
            pltpu.VMEM((2, m, n_per), jnp.bfloat16),
            pltpu.VMEM((5, m, n_per), jnp.float32),
            pltpu.VMEM((N_DEV, m), jnp.float32),
            pltpu.SemaphoreType.DMA((2, SUB)),
            pltpu.SemaphoreType.DMA((2, SUB)),
            pltpu.SemaphoreType.DMA((2, SUB)),
            pltpu.SemaphoreType.DMA((2, SUB)),
            pltpu.SemaphoreType.DMA((N_DEV,)),
            pltpu.SemaphoreType.DMA((N_DEV,)),
            pltpu.SemaphoreType.DMA((5,)),
        ],
        compiler_params=pltpu.CompilerParams(collective_id=0),
    )(x, W)
